# Initial kernel scaffold; baseline (speedup 1.0000x reference)
import jax
import jax.numpy as jnp
from jax.experimental import pallas as pl


def kernel(hidden_states, gate_weight, expert_gate_w, expert_up_w, expert_down_w, shared_gate_w, shared_up_w, shared_down_w):
    raise NotImplementedError("write your pallas kernel here")


# speedup vs baseline: 2.8391x; 2.8391x over previous
import functools

import jax
import jax.numpy as jnp
from jax import lax
from jax.experimental import pallas as pl
from jax.experimental.pallas import tpu as pltpu

S = 2048
D = 1024
E = 16
FF = 2048
C = 256
NF = 2
FB = FF // NF
SHARED_FF = 2 * D
NSF = 2
SFB = SHARED_FF // NSF
NSB = 2
SB = S // NSB


def _cumsum0(x):
    n = x.shape[0]
    k = 1
    while k < n:
        pad = jnp.zeros((k,) + x.shape[1:], x.dtype)
        x = x + jnp.concatenate([pad, x[:-k]], axis=0)
        k *= 2
    return x


def _gelu_exact(x):
    return x * 0.5 * (1.0 + lax.erf(x * 0.7071067811865476))


def _gating_body(x_ref, gw_ref, wsel_ref, pos_ref, xbf_ref):
    x = x_ref[...]
    logits = lax.dot_general(x, gw_ref[...], (((1,), (1,)), ((), ())),
                             preferred_element_type=jnp.float32)
    m = jnp.max(logits, axis=1, keepdims=True)
    ex = jnp.exp(logits - m)
    scores = ex / jnp.sum(ex, axis=1, keepdims=True)

    bits = lax.bitcast_convert_type(scores, jnp.int32)
    lo0 = jnp.full((1, E), -1, jnp.int32)
    hi0 = jnp.max(bits, axis=0, keepdims=True) + 1

    def body(_, carry):
        lo, hi = carry
        mid = lo + (hi - lo) // 2
        cnt = jnp.sum((bits > mid).astype(jnp.int32), axis=0, keepdims=True)
        pred = cnt >= C
        return jnp.where(pred, mid, lo), jnp.where(pred, hi, mid)

    _, kth = lax.fori_loop(0, 31, body, (lo0, hi0))
    gt = bits > kth
    eq = bits == kth
    cnt_gt = jnp.sum(gt.astype(jnp.int32), axis=0, keepdims=True)
    need = C - cnt_gt
    eq_i = eq.astype(jnp.int32)
    rank_excl = _cumsum0(eq_i) - eq_i
    sel = gt | (eq & (rank_excl < need))
    slot = _cumsum0(sel.astype(jnp.int32)) - 1
    pos_ref[...] = jnp.where(sel, slot, -1)
    wsel_ref[...] = jnp.where(sel, scores, 0.0)
    xbf_ref[...] = x.astype(jnp.bfloat16)


def _shared_body(xbf_ref, sg_ref, su_ref, sd_ref, out_ref):
    f = pl.program_id(1)
    xbf = xbf_ref[...]
    sg = sg_ref[...].astype(jnp.bfloat16)
    su = su_ref[...].astype(jnp.bfloat16)
    sd = sd_ref[...].astype(jnp.bfloat16)
    nt = (((1,), (1,)), ((), ()))
    g = lax.dot_general(xbf, sg, nt, preferred_element_type=jnp.float32)
    u = lax.dot_general(xbf, su, nt, preferred_element_type=jnp.float32)
    h = (_gelu_exact(g) * u).astype(jnp.bfloat16)
    part = lax.dot_general(h, sd, nt, preferred_element_type=jnp.float32)

    @pl.when(f == 0)
    def _():
        out_ref[...] = part

    @pl.when(f != 0)
    def _():
        out_ref[...] += part


def _expert_body(pos_ref, wsel_ref, xbf_ref, wg_ref, wu_ref, wd_ref, sh_ref,
                 out_ref, tok_ref, yacc_ref):
    e = pl.program_id(0)
    f = pl.program_id(1)

    @pl.when((e == 0) & (f == 0))
    def _():
        out_ref[...] = sh_ref[...]

    lane = lax.broadcasted_iota(jnp.int32, (S, E), 1)
    esel = lane == e
    pos_e = jnp.sum(jnp.where(esel, pos_ref[...], 0), axis=1, keepdims=True)
    slot_iota = lax.broadcasted_iota(jnp.int32, (S, C), 1)
    ps_bool = pos_e == slot_iota

    @pl.when(f == 0)
    def _():
        tok_ref[...] = lax.dot_general(
            ps_bool.astype(jnp.bfloat16), xbf_ref[...],
            (((0,), (0,)), ((), ())),
            preferred_element_type=jnp.float32).astype(jnp.bfloat16)

    tok = tok_ref[...]
    wg = wg_ref[0].astype(jnp.bfloat16)
    wu = wu_ref[0].astype(jnp.bfloat16)
    wd = wd_ref[0].astype(jnp.bfloat16)
    nt = (((1,), (1,)), ((), ()))
    g = lax.dot_general(tok, wg, nt, preferred_element_type=jnp.float32)
    u = lax.dot_general(tok, wu, nt, preferred_element_type=jnp.float32)
    h = (_gelu_exact(g) * u).astype(jnp.bfloat16)
    part = lax.dot_general(h, wd, nt, preferred_element_type=jnp.float32)

    @pl.when(f == 0)
    def _():
        yacc_ref[...] = part

    @pl.when(f != 0)
    def _():
        yacc_ref[...] += part

    @pl.when(f == NF - 1)
    def _():
        w_e = jnp.sum(jnp.where(esel, wsel_ref[...], 0.0), axis=1,
                      keepdims=True)
        psw = jnp.where(ps_bool, w_e, 0.0).astype(jnp.bfloat16)
        ybf = yacc_ref[...].astype(jnp.bfloat16)
        out_ref[...] += lax.dot_general(
            psw, ybf, (((1,), (0,)), ((), ())),
            preferred_element_type=jnp.float32)


def _gating(x, gate_weight, interpret=False):
    return pl.pallas_call(
        _gating_body,
        out_shape=(
            jax.ShapeDtypeStruct((S, E), jnp.float32),
            jax.ShapeDtypeStruct((S, E), jnp.int32),
            jax.ShapeDtypeStruct((S, D), jnp.bfloat16),
        ),
        interpret=interpret,
    )(x, gate_weight)


def _shared(xbf, sgw, suw, sdw, interpret=False):
    return pl.pallas_call(
        _shared_body,
        grid=(NSB, NSF),
        in_specs=[
            pl.BlockSpec((SB, D), lambda s, f: (s, 0)),
            pl.BlockSpec((SFB, D), lambda s, f: (f, 0)),
            pl.BlockSpec((SFB, D), lambda s, f: (f, 0)),
            pl.BlockSpec((D, SFB), lambda s, f: (0, f)),
        ],
        out_specs=pl.BlockSpec((SB, D), lambda s, f: (s, 0)),
        out_shape=jax.ShapeDtypeStruct((S, D), jnp.float32),
        interpret=interpret,
    )(xbf, sgw, suw, sdw)


def _experts(pos, wsel, xbf, weg, weu, wed, sh, interpret=False):
    return pl.pallas_call(
        _expert_body,
        grid=(E, NF),
        in_specs=[
            pl.BlockSpec((S, E), lambda e, f: (0, 0)),
            pl.BlockSpec((S, E), lambda e, f: (0, 0)),
            pl.BlockSpec((S, D), lambda e, f: (0, 0)),
            pl.BlockSpec((1, FB, D), lambda e, f: (e, f, 0)),
            pl.BlockSpec((1, FB, D), lambda e, f: (e, f, 0)),
            pl.BlockSpec((1, D, FB), lambda e, f: (e, 0, f)),
            pl.BlockSpec((S, D), lambda e, f: (0, 0)),
        ],
        out_specs=pl.BlockSpec((S, D), lambda e, f: (0, 0)),
        out_shape=jax.ShapeDtypeStruct((S, D), jnp.float32),
        scratch_shapes=[
            pltpu.VMEM((C, D), jnp.bfloat16),
            pltpu.VMEM((C, D), jnp.float32),
        ],
        interpret=interpret,
    )(pos, wsel, xbf, weg, weu, wed, sh)


def kernel(hidden_states, gate_weight, expert_gate_w, expert_up_w,
           expert_down_w, shared_gate_w, shared_up_w, shared_down_w,
           interpret=False):
    b, s, d = hidden_states.shape
    x = hidden_states.reshape(s, d)
    wsel, pos, xbf = _gating(x, gate_weight, interpret=interpret)
    sh = _shared(xbf, shared_gate_w, shared_up_w, shared_down_w,
                 interpret=interpret)
    out = _experts(pos, wsel, xbf, expert_gate_w, expert_up_w, expert_down_w,
                   sh, interpret=interpret)
    return out.reshape(b, s, d)
